# baseline (speedup 1.0000x reference)
import functools
import math

import jax
import jax.numpy as jnp
from jax import lax
from jax.experimental import pallas as pl
from jax.experimental.pallas import tpu as pltpu
from jax.experimental.pallas import tpu_sc as plsc

N = 10000
NPAD = 10240
E = 320000
EP = 320512
NC = 2
NS = 16
CH = 128
NCHUNK = EP // CH
KMAX = 160
RPT = NPAD // NS
D = 64
BNS = 1.0 / math.sqrt(1.0 + 1e-5)
RB = 5000
GRID = N // RB

_f32 = jnp.float32


def _load_idx(ei_hbm, row, s, idx_v):
    @pl.when(s < NS - 1)
    def _():
        pltpu.sync_copy(ei_hbm.at[row, pl.ds(s * KMAX, KMAX)], idx_v)

    @pl.when(s == NS - 1)
    def _():
        pltpu.sync_copy(ei_hbm.at[row, pl.ds(s * KMAX, 104)],
                        idx_v.at[pl.ds(0, 104)])


def _sc_deg_body(sp_hbm, tr_hbm, deg_hbm, idx_v, part_v, buf_v, out_v,
                 stage_v):
    c = lax.axis_index("c")
    s = lax.axis_index("s")
    cnt = jnp.where(s < NS - 1, KMAX, 104)

    @pl.when(c == 0)
    def _():
        _load_idx(sp_hbm, 1, s, idx_v)

    @pl.when(c == 1)
    def _():
        _load_idx(tr_hbm, 1, s, idx_v)

    zeros16 = jnp.zeros((16,), _f32)
    ones16 = jnp.ones((16,), _f32)

    @pl.loop(0, NPAD // 16)
    def _zero(i):
        part_v[pl.ds(i * 16, 16)] = zeros16

    @pl.loop(0, cnt)
    def _count(j):
        for k in range(CH // 16):
            idx = idx_v[j, pl.ds(k * 16, 16)]
            plsc.addupdate_scatter(part_v, [idx], ones16)

    pltpu.sync_copy(part_v, stage_v.at[s])
    plsc.subcore_barrier()
    base = s * RPT
    for t in range(NS):
        pltpu.sync_copy(stage_v.at[t, pl.ds(base, RPT)], buf_v.at[t])

    @pl.loop(0, RPT // 16)
    def _sum(k):
        acc = buf_v[0, pl.ds(k * 16, 16)]
        for t in range(1, NS):
            acc = acc + buf_v[t, pl.ds(k * 16, 16)]
        out_v[pl.ds(k * 16, 16)] = acc

    pltpu.sync_copy(out_v, deg_hbm.at[c, pl.ds(base, RPT)])


_sc_deg = pl.kernel(
    _sc_deg_body,
    out_type=jax.ShapeDtypeStruct((NC, NPAD), _f32),
    compiler_params=pltpu.CompilerParams(needs_layout_passes=False),
    mesh=plsc.VectorSubcoreMesh(core_axis_name="c", subcore_axis_name="s"),
    scratch_types=[
        pltpu.VMEM((KMAX, CH), jnp.int32),
        pltpu.VMEM((NPAD,), _f32),
        pltpu.VMEM((NS, RPT), _f32),
        pltpu.VMEM((RPT,), _f32),
        pltpu.VMEM_SHARED((NS, NPAD), _f32),
    ],
)


def _sc_msg_body(h2_hbm, sp_hbm, tr_hbm, ss_hbm, st_hbm,
                 srcv, dstv, rows0, rows1, rows2, rows3, shared,
                 semg0, semg1, sems0, sems1):
    c = lax.axis_index("c")
    s = lax.axis_index("s")
    ng = jnp.where(s < NS - 1, KMAX // 2, 104 // 2)

    @pl.when(c == 0)
    def _():
        _load_idx(sp_hbm, 0, s, srcv)
        _load_idx(sp_hbm, 1, s, dstv)

    @pl.when(c == 1)
    def _():
        _load_idx(tr_hbm, 0, s, srcv)
        _load_idx(tr_hbm, 1, s, dstv)

    zeros16 = jnp.zeros((16,), _f32)

    @pl.loop(0, CH)
    def _zr(i):
        for k in range(D // 16):
            rows0[i, pl.ds(k * 16, 16)] = zeros16

    base = s * RPT
    for b in range(RPT // CH):
        pltpu.sync_copy(rows0, shared.at[pl.ds(base + b * CH, CH)])
    plsc.subcore_barrier()

    semg = (semg0, semg1)
    sems = (sems0, sems1)
    bufs = ((rows0, rows1), (rows2, rows3))

    def fire_g(slot, g):
        for b in range(2):
            pltpu.async_copy(h2_hbm.at[srcv.at[g * 2 + b]],
                             bufs[slot][b], semg[slot])

    def wait_g(slot, g):
        for b in range(2):
            pltpu.make_async_copy(h2_hbm.at[srcv.at[g * 2 + b]],
                                  bufs[slot][b], semg[slot]).wait()

    def fire_s(slot, g):
        for b in range(2):
            pltpu.async_copy(bufs[slot][b],
                             shared.at[dstv.at[g * 2 + b]], sems[slot],
                             add=True)

    def wait_s(slot, g):
        for b in range(2):
            pltpu.make_async_copy(bufs[slot][b],
                                  shared.at[dstv.at[g * 2 + b]],
                                  sems[slot]).wait()

    fire_g(0, 0)

    @pl.loop(0, ng // 2)
    def _mp(i):
        g = i * 2

        @pl.when(i > 0)
        def _():
            wait_s(1, g - 1)

        fire_g(1, g + 1)
        wait_g(0, g)
        fire_s(0, g)
        wait_s(0, g)

        @pl.when(g + 2 < ng)
        def _():
            fire_g(0, g + 2)

        wait_g(1, g + 1)
        fire_s(1, g + 1)

    wait_s(1, ng - 1)
    plsc.subcore_barrier()

    def copy_out(out_hbm):
        for b in range(3):
            pltpu.sync_copy(shared.at[pl.ds(base + b * CH, CH)], rows0)
            pltpu.sync_copy(rows0, out_hbm.at[pl.ds(base + b * CH, CH)])

        @pl.when(s < NS - 1)
        def _():
            for b in range(3, 5):
                pltpu.sync_copy(shared.at[pl.ds(base + b * CH, CH)], rows0)
                pltpu.sync_copy(rows0, out_hbm.at[pl.ds(base + b * CH, CH)])

        @pl.when(s == NS - 1)
        def _():
            pltpu.sync_copy(shared.at[pl.ds(base + 3 * CH, 16)],
                            rows0.at[pl.ds(0, 16)])
            pltpu.sync_copy(rows0.at[pl.ds(0, 16)],
                            out_hbm.at[pl.ds(base + 3 * CH, 16)])

    @pl.when(c == 0)
    def _():
        copy_out(ss_hbm)

    @pl.when(c == 1)
    def _():
        copy_out(st_hbm)


_sc_msg = pl.kernel(
    _sc_msg_body,
    out_type=(jax.ShapeDtypeStruct((N, D), _f32),
              jax.ShapeDtypeStruct((N, D), _f32)),
    compiler_params=pltpu.CompilerParams(use_tc_tiling_on_sc=False),
    mesh=plsc.VectorSubcoreMesh(core_axis_name="c", subcore_axis_name="s"),
    scratch_types=[
        pltpu.VMEM((KMAX, CH), jnp.int32),
        pltpu.VMEM((KMAX, CH), jnp.int32),
        pltpu.VMEM((CH, D), _f32),
        pltpu.VMEM((CH, D), _f32),
        pltpu.VMEM((CH, D), _f32),
        pltpu.VMEM((CH, D), _f32),
        pltpu.VMEM_SHARED((NPAD, D), _f32),
        pltpu.SemaphoreType.DMA,
        pltpu.SemaphoreType.DMA,
        pltpu.SemaphoreType.DMA,
        pltpu.SemaphoreType.DMA,
    ],
)


def _tc_enc_body(ctx_r, vis_r, degc_r,
                 cW1, cb1, cg1, cbe1, cW2, cb2, cg2, cbe2,
                 tW1, tb1, tW2, tb2, sW, trW,
                 fus_o, h2_o, dinv_o):
    x = ctx_r[...]
    c1 = jnp.maximum(
        (jnp.dot(x, cW1[...], preferred_element_type=_f32) + cb1[...])
        * (cg1[...] * BNS) + cbe1[...], 0.0)
    c2 = jnp.maximum(
        (jnp.dot(c1, cW2[...], preferred_element_type=_f32) + cb2[...])
        * (cg2[...] * BNS) + cbe2[...], 0.0)
    t1 = jnp.maximum(
        jnp.dot(vis_r[...], tW1[...], preferred_element_type=_f32) + tb1[...],
        0.0)
    t2 = jnp.dot(t1, tW2[...], preferred_element_type=_f32) + tb2[...]
    fus = jnp.concatenate([c2, t2], axis=1)
    fus_o[...] = fus
    dinv = lax.rsqrt(degc_r[...] + 1.0)
    dinv_o[...] = dinv
    h2_o[0] = jnp.dot(fus, sW[...], preferred_element_type=_f32) * dinv[:, 0:1]
    h2_o[1] = jnp.dot(fus, trW[...], preferred_element_type=_f32) * dinv[:, 1:2]


def _full(shape):
    nd = len(shape)
    return pl.BlockSpec(shape, lambda r: (0,) * nd)


def _rows(w):
    return pl.BlockSpec((RB, w), lambda r: (r, 0))


def _tc_enc(ctx, vis, degc, *ws):
    w_specs = [_full(w.shape) for w in ws]
    return pl.pallas_call(
        _tc_enc_body,
        grid=(GRID,),
        in_specs=[_rows(128), _rows(64), _rows(2)] + w_specs,
        out_specs=[_rows(192),
                   pl.BlockSpec((2, RB, D), lambda r: (0, r, 0)),
                   _rows(2)],
        out_shape=[jax.ShapeDtypeStruct((N, 192), _f32),
                   jax.ShapeDtypeStruct((2, N, D), _f32),
                   jax.ShapeDtypeStruct((N, 2), _f32)],
    )(ctx, vis, degc, *ws)


def _tc_post_body(ss_r, st_r, hsp_r, htp_r, fus_r, dinv_r,
                  sb, trb, fW, fb, hW1, hb1, hg1, hbe1, hW2, hb2, hW3, hb3,
                  out_o, h_o):
    dinv = dinv_r[...]
    hs = jnp.maximum(dinv[:, 0:1] * (ss_r[...] + hsp_r[0]) + sb[...], 0.0)
    ht = jnp.maximum(dinv[:, 1:2] * (st_r[...] + htp_r[0]) + trb[...], 0.0)
    hcat = jnp.concatenate([hs, ht], axis=1)
    h = jnp.maximum(
        jnp.dot(hcat, fW[...], preferred_element_type=_f32) + fb[...], 0.0)
    h_o[...] = h
    zin = jnp.concatenate([h, fus_r[...]], axis=1)
    z1 = jnp.maximum(
        (jnp.dot(zin, hW1[...], preferred_element_type=_f32) + hb1[...])
        * (hg1[...] * BNS) + hbe1[...], 0.0)
    z2 = jnp.maximum(
        jnp.dot(z1, hW2[...], preferred_element_type=_f32) + hb2[...], 0.0)
    out_o[...] = jnp.dot(z2, hW3[...], preferred_element_type=_f32) + hb3[...]


def _tc_post(ss, st, h2, fus, dinvc, *ws):
    w_specs = [_full(w.shape) for w in ws]
    return pl.pallas_call(
        _tc_post_body,
        grid=(GRID,),
        in_specs=[_rows(D), _rows(D),
                  pl.BlockSpec((1, RB, D), lambda r: (0, r, 0)),
                  pl.BlockSpec((1, RB, D), lambda r: (1, r, 0)),
                  _rows(192), _rows(2)] + w_specs,
        out_specs=[_rows(D), _rows(D)],
        out_shape=[jax.ShapeDtypeStruct((N, D), _f32),
                   jax.ShapeDtypeStruct((N, D), _f32)],
    )(ss, st, h2, h2, fus, dinvc, *ws)


def kernel(ctx, vis_tgt, sp_ei, tr_ei,
           ctx_W1, ctx_b1, ctx_g1, ctx_be1, ctx_W2, ctx_b2, ctx_g2, ctx_be2,
           tgt_W1, tgt_b1, tgt_W2, tgt_b2,
           spa_W, spa_b, tra_W, tra_b, fuse_W, fuse_b,
           head_W1, head_b1, head_g1, head_be1, head_W2, head_b2,
           head_W3, head_b3):
    i32 = jnp.int32
    pad2 = jnp.stack([jnp.arange(EP - E, dtype=i32),
                      N + jnp.arange(EP - E, dtype=i32) % (NPAD - N)])
    sp3 = jnp.concatenate([sp_ei, pad2], axis=1).reshape(2, NCHUNK, CH)
    off = jnp.stack([jnp.full((E,), N, i32), jnp.zeros((E,), i32)])
    tr3 = jnp.concatenate([tr_ei + off, pad2],
                          axis=1).reshape(2, NCHUNK, CH)

    deg = _sc_deg(sp3, tr3)
    degc = jnp.transpose(deg[:, :N])

    fus, h2, dinvc = _tc_enc(
        ctx, vis_tgt, degc,
        ctx_W1, ctx_b1, ctx_g1, ctx_be1, ctx_W2, ctx_b2, ctx_g2, ctx_be2,
        tgt_W1, tgt_b1, tgt_W2, tgt_b2, spa_W, tra_W)

    ss, st = _sc_msg(h2.reshape(2 * N, D), sp3, tr3)

    out, h = _tc_post(
        ss, st, h2, fus, dinvc,
        spa_b, tra_b, fuse_W, fuse_b,
        head_W1, head_b1, head_g1, head_be1, head_W2, head_b2,
        head_W3, head_b3)
    return out, h


# speedup vs baseline: 1.0040x; 1.0040x over previous
import math

import jax
import jax.numpy as jnp
from jax import lax
from jax.experimental import pallas as pl
from jax.experimental.pallas import tpu as pltpu
from jax.experimental.pallas import tpu_sc as plsc

N = 10000
NPAD = 10240
E = 320000
EP = 320512
NC = 2
NS = 16
CH = 128
NCHUNK = EP // CH
KMAX = 160
RPT = NPAD // NS
D = 64
BNS = 1.0 / math.sqrt(1.0 + 1e-5)
RB = 2000
GRID = N // RB

_f32 = jnp.float32


def _load_idx(ei_hbm, row, s, idx_v):
    @pl.when(s < NS - 1)
    def _():
        pltpu.sync_copy(ei_hbm.at[row, pl.ds(s * KMAX, KMAX)], idx_v)

    @pl.when(s == NS - 1)
    def _():
        pltpu.sync_copy(ei_hbm.at[row, pl.ds(s * KMAX, 104)],
                        idx_v.at[pl.ds(0, 104)])


def _sc_deg_body(sp_hbm, tr_hbm, deg_hbm, idx_v, part_v, buf_v, out_v,
                 stage_v):
    c = lax.axis_index("c")
    s = lax.axis_index("s")
    cnt = jnp.where(s < NS - 1, KMAX, 104)

    @pl.when(c == 0)
    def _():
        _load_idx(sp_hbm, 1, s, idx_v)

    @pl.when(c == 1)
    def _():
        _load_idx(tr_hbm, 1, s, idx_v)

    zeros16 = jnp.zeros((16,), _f32)
    ones16 = jnp.ones((16,), _f32)

    @pl.loop(0, NPAD // 16)
    def _zero(i):
        part_v[pl.ds(i * 16, 16)] = zeros16

    @pl.loop(0, cnt)
    def _count(j):
        for k in range(CH // 16):
            idx = idx_v[j, pl.ds(k * 16, 16)]
            plsc.addupdate_scatter(part_v, [idx], ones16)

    pltpu.sync_copy(part_v, stage_v.at[s])
    plsc.subcore_barrier()
    base = s * RPT
    for t in range(NS):
        pltpu.sync_copy(stage_v.at[t, pl.ds(base, RPT)], buf_v.at[t])

    @pl.loop(0, RPT // 16)
    def _sum(k):
        acc = buf_v[0, pl.ds(k * 16, 16)]
        for t in range(1, NS):
            acc = acc + buf_v[t, pl.ds(k * 16, 16)]
        out_v[pl.ds(k * 16, 16)] = acc

    pltpu.sync_copy(out_v, deg_hbm.at[c, pl.ds(base, RPT)])


_sc_deg = pl.kernel(
    _sc_deg_body,
    out_type=jax.ShapeDtypeStruct((NC, NPAD), _f32),
    compiler_params=pltpu.CompilerParams(needs_layout_passes=False),
    mesh=plsc.VectorSubcoreMesh(core_axis_name="c", subcore_axis_name="s"),
    scratch_types=[
        pltpu.VMEM((KMAX, CH), jnp.int32),
        pltpu.VMEM((NPAD,), _f32),
        pltpu.VMEM((NS, RPT), _f32),
        pltpu.VMEM((RPT,), _f32),
        pltpu.VMEM_SHARED((NS, NPAD), _f32),
    ],
)


def _sc_msg_body(h2_hbm, sp_hbm, tr_hbm, ss_hbm, st_hbm,
                 srcv, dstv, rows0, rows1, rows2, rows3, shared,
                 semg0, semg1, sems0, sems1):
    c = lax.axis_index("c")
    s = lax.axis_index("s")
    ng = jnp.where(s < NS - 1, KMAX // 2, 104 // 2)

    @pl.when(c == 0)
    def _():
        _load_idx(sp_hbm, 0, s, srcv)
        _load_idx(sp_hbm, 1, s, dstv)

    @pl.when(c == 1)
    def _():
        _load_idx(tr_hbm, 0, s, srcv)
        _load_idx(tr_hbm, 1, s, dstv)

    zeros16 = jnp.zeros((16,), _f32)

    @pl.loop(0, CH)
    def _zr(i):
        for k in range(D // 16):
            rows0[i, pl.ds(k * 16, 16)] = zeros16

    base = s * RPT
    for b in range(RPT // CH):
        pltpu.sync_copy(rows0, shared.at[pl.ds(base + b * CH, CH)])
    plsc.subcore_barrier()

    semg = (semg0, semg1)
    sems = (sems0, sems1)
    bufs = ((rows0, rows1), (rows2, rows3))

    def fire_g(slot, g):
        for b in range(2):
            pltpu.async_copy(h2_hbm.at[srcv.at[g * 2 + b]],
                             bufs[slot][b], semg[slot])

    def wait_g(slot, g):
        for b in range(2):
            pltpu.make_async_copy(h2_hbm.at[srcv.at[g * 2 + b]],
                                  bufs[slot][b], semg[slot]).wait()

    def fire_s(slot, g):
        for b in range(2):
            pltpu.async_copy(bufs[slot][b],
                             shared.at[dstv.at[g * 2 + b]], sems[slot],
                             add=True)

    def wait_s(slot, g):
        for b in range(2):
            pltpu.make_async_copy(bufs[slot][b],
                                  shared.at[dstv.at[g * 2 + b]],
                                  sems[slot]).wait()

    fire_g(0, 0)

    @pl.loop(0, ng // 2)
    def _mp(i):
        g = i * 2

        @pl.when(i > 0)
        def _():
            wait_s(1, g - 1)

        fire_g(1, g + 1)
        wait_g(0, g)
        fire_s(0, g)
        wait_s(0, g)

        @pl.when(g + 2 < ng)
        def _():
            fire_g(0, g + 2)

        wait_g(1, g + 1)
        fire_s(1, g + 1)

    wait_s(1, ng - 1)
    plsc.subcore_barrier()

    def copy_out(out_hbm):
        for b in range(3):
            pltpu.sync_copy(shared.at[pl.ds(base + b * CH, CH)], rows0)
            pltpu.sync_copy(rows0, out_hbm.at[pl.ds(base + b * CH, CH)])

        @pl.when(s < NS - 1)
        def _():
            for b in range(3, 5):
                pltpu.sync_copy(shared.at[pl.ds(base + b * CH, CH)], rows0)
                pltpu.sync_copy(rows0, out_hbm.at[pl.ds(base + b * CH, CH)])

        @pl.when(s == NS - 1)
        def _():
            pltpu.sync_copy(shared.at[pl.ds(base + 3 * CH, 16)],
                            rows0.at[pl.ds(0, 16)])
            pltpu.sync_copy(rows0.at[pl.ds(0, 16)],
                            out_hbm.at[pl.ds(base + 3 * CH, 16)])

    @pl.when(c == 0)
    def _():
        copy_out(ss_hbm)

    @pl.when(c == 1)
    def _():
        copy_out(st_hbm)


_sc_msg = pl.kernel(
    _sc_msg_body,
    out_type=(jax.ShapeDtypeStruct((N, D), _f32),
              jax.ShapeDtypeStruct((N, D), _f32)),
    compiler_params=pltpu.CompilerParams(use_tc_tiling_on_sc=False),
    mesh=plsc.VectorSubcoreMesh(core_axis_name="c", subcore_axis_name="s"),
    scratch_types=[
        pltpu.VMEM((KMAX, CH), jnp.int32),
        pltpu.VMEM((KMAX, CH), jnp.int32),
        pltpu.VMEM((CH, D), _f32),
        pltpu.VMEM((CH, D), _f32),
        pltpu.VMEM((CH, D), _f32),
        pltpu.VMEM((CH, D), _f32),
        pltpu.VMEM_SHARED((NPAD, D), _f32),
        pltpu.SemaphoreType.DMA,
        pltpu.SemaphoreType.DMA,
        pltpu.SemaphoreType.DMA,
        pltpu.SemaphoreType.DMA,
    ],
)


def _tc_enc_body(ctx_r, vis_r, degc_r,
                 cW1, cb1, cg1, cbe1, cW2, cb2, cg2, cbe2,
                 tW1, tb1, tW2, tb2, sW, trW,
                 fus_o, h2_o, dinv_o):
    x = ctx_r[...]
    c1 = jnp.maximum(
        (jnp.dot(x, cW1[...], preferred_element_type=_f32) + cb1[...])
        * (cg1[...] * BNS) + cbe1[...], 0.0)
    c2 = jnp.maximum(
        (jnp.dot(c1, cW2[...], preferred_element_type=_f32) + cb2[...])
        * (cg2[...] * BNS) + cbe2[...], 0.0)
    t1 = jnp.maximum(
        jnp.dot(vis_r[...], tW1[...], preferred_element_type=_f32) + tb1[...],
        0.0)
    t2 = jnp.dot(t1, tW2[...], preferred_element_type=_f32) + tb2[...]
    fus = jnp.concatenate([c2, t2], axis=1)
    fus_o[...] = fus
    dinv = lax.rsqrt(degc_r[...] + 1.0)
    dinv_o[...] = dinv
    h2_o[0] = jnp.dot(fus, sW[...], preferred_element_type=_f32) * dinv[:, 0:1]
    h2_o[1] = jnp.dot(fus, trW[...], preferred_element_type=_f32) * dinv[:, 1:2]


def _full(shape):
    nd = len(shape)
    return pl.BlockSpec(shape, lambda r: (0,) * nd)


def _rows(w):
    return pl.BlockSpec((RB, w), lambda r: (r, 0))


def _tc_enc(ctx, vis, degc, *ws):
    w_specs = [_full(w.shape) for w in ws]
    return pl.pallas_call(
        _tc_enc_body,
        grid=(GRID,),
        in_specs=[_rows(128), _rows(64), _rows(2)] + w_specs,
        out_specs=[_rows(192),
                   pl.BlockSpec((2, RB, D), lambda r: (0, r, 0)),
                   _rows(2)],
        out_shape=[jax.ShapeDtypeStruct((N, 192), _f32),
                   jax.ShapeDtypeStruct((2, N, D), _f32),
                   jax.ShapeDtypeStruct((N, 2), _f32)],
    )(ctx, vis, degc, *ws)


def _tc_post_body(ss_r, st_r, hsp_r, htp_r, fus_r, dinv_r,
                  sb, trb, fW, fb, hW1, hb1, hg1, hbe1, hW2, hb2, hW3, hb3,
                  out_o, h_o):
    dinv = dinv_r[...]
    hs = jnp.maximum(dinv[:, 0:1] * (ss_r[...] + hsp_r[0]) + sb[...], 0.0)
    ht = jnp.maximum(dinv[:, 1:2] * (st_r[...] + htp_r[0]) + trb[...], 0.0)
    hcat = jnp.concatenate([hs, ht], axis=1)
    h = jnp.maximum(
        jnp.dot(hcat, fW[...], preferred_element_type=_f32) + fb[...], 0.0)
    h_o[...] = h
    zin = jnp.concatenate([h, fus_r[...]], axis=1)
    z1 = jnp.maximum(
        (jnp.dot(zin, hW1[...], preferred_element_type=_f32) + hb1[...])
        * (hg1[...] * BNS) + hbe1[...], 0.0)
    z2 = jnp.maximum(
        jnp.dot(z1, hW2[...], preferred_element_type=_f32) + hb2[...], 0.0)
    out_o[...] = jnp.dot(z2, hW3[...], preferred_element_type=_f32) + hb3[...]


def _tc_post(ss, st, h2, fus, dinvc, *ws):
    w_specs = [_full(w.shape) for w in ws]
    return pl.pallas_call(
        _tc_post_body,
        grid=(GRID,),
        in_specs=[_rows(D), _rows(D),
                  pl.BlockSpec((1, RB, D), lambda r: (0, r, 0)),
                  pl.BlockSpec((1, RB, D), lambda r: (1, r, 0)),
                  _rows(192), _rows(2)] + w_specs,
        out_specs=[_rows(D), _rows(D)],
        out_shape=[jax.ShapeDtypeStruct((N, D), _f32),
                   jax.ShapeDtypeStruct((N, D), _f32)],
    )(ss, st, h2, h2, fus, dinvc, *ws)


def kernel(ctx, vis_tgt, sp_ei, tr_ei,
           ctx_W1, ctx_b1, ctx_g1, ctx_be1, ctx_W2, ctx_b2, ctx_g2, ctx_be2,
           tgt_W1, tgt_b1, tgt_W2, tgt_b2,
           spa_W, spa_b, tra_W, tra_b, fuse_W, fuse_b,
           head_W1, head_b1, head_g1, head_be1, head_W2, head_b2,
           head_W3, head_b3):
    i32 = jnp.int32
    pad2 = jnp.stack([jnp.arange(EP - E, dtype=i32),
                      N + jnp.arange(EP - E, dtype=i32) % (NPAD - N)])
    sp3 = jnp.concatenate([sp_ei, pad2], axis=1).reshape(2, NCHUNK, CH)
    off = jnp.stack([jnp.full((E,), N, i32), jnp.zeros((E,), i32)])
    tr3 = jnp.concatenate([tr_ei + off, pad2],
                          axis=1).reshape(2, NCHUNK, CH)

    deg = _sc_deg(sp3, tr3)
    degc = jnp.transpose(deg[:, :N])

    fus, h2, dinvc = _tc_enc(
        ctx, vis_tgt, degc,
        ctx_W1, ctx_b1, ctx_g1, ctx_be1, ctx_W2, ctx_b2, ctx_g2, ctx_be2,
        tgt_W1, tgt_b1, tgt_W2, tgt_b2, spa_W, tra_W)

    ss, st = _sc_msg(h2.reshape(2 * N, D), sp3, tr3)

    out, h = _tc_post(
        ss, st, h2, fus, dinvc,
        spa_b, tra_b, fuse_W, fuse_b,
        head_W1, head_b1, head_g1, head_be1, head_W2, head_b2,
        head_W3, head_b3)
    return out, h
